# baseline (speedup 1.0000x reference)
import functools

import jax
import jax.numpy as jnp
from jax import lax
from jax.experimental import pallas as pl
from jax.experimental.pallas import tpu as pltpu
from jax.experimental.pallas import tpu_sc as plsc

_NW = 32
_RPW = 16384 // _NW
_DR = 32
_YC = 128


def _body(x_hbm, y_hbm, z_hbm, xo_hbm, yo_hbm, zo_hbm,
          dbuf, ybuf, out_v, dsem, ysem, osem, yosem):
    c = lax.axis_index("c")
    s = lax.axis_index("s")
    w = s * 2 + c

    y0 = w * _RPW
    y_in = [None] * 4

    def start_y(k):
        y_in[k] = pltpu.async_copy(
            y_hbm.at[pl.ds(y0 + k * _YC, _YC), pl.ds(0, 128)],
            ybuf.at[k % 2],
            ysem.at[k % 2],
        )

    start_y(0)
    start_y(1)

    p = w // 16
    q = w % 16
    src_row = jnp.where(p == 0, 8, 5)
    pairs = []
    for i in range(4):
        r = pl.ds(q * 128 + i * _DR, _DR)
        pairs.append((x_hbm.at[src_row, r], xo_hbm.at[p, r]))
    for i in range(2):
        r = pl.ds(w * 64 + i * _DR, _DR)
        pairs.append((z_hbm.at[2, r], zo_hbm.at[0, r]))

    d_in = [None] * 6
    d_out = [None] * 6

    def start_d(i):
        d_in[i] = pltpu.async_copy(pairs[i][0], dbuf.at[i % 2], dsem.at[i % 2])

    start_d(0)
    for i in range(6):
        if i + 1 < 6:
            if i >= 1:
                d_out[i - 1].wait()
            start_d(i + 1)
        d_in[i].wait()
        d_out[i] = pltpu.async_copy(dbuf.at[i % 2], pairs[i][1], osem)

    lanes = lax.iota(jnp.int32, 16)
    for k in range(4):
        y_in[k].wait()
        for j in range(16):
            k16 = j * 16 + lanes
            out_v[k * 16 + j] = plsc.load_gather(
                ybuf.at[k % 2], [k16 >> 1, 1 - (k16 & 1)]
            )
        if k + 2 < 4:
            start_y(k + 2)
    yo_dma = pltpu.async_copy(out_v, yo_hbm.at[w], yosem)

    d_out[4].wait()
    d_out[5].wait()
    yo_dma.wait()


def kernel(x, y, z):
    x2 = x.reshape(16, 2048, 1024)
    y2 = y.reshape(16384, 2048)
    z2 = z.reshape(8, 2048, 1024)

    mesh = plsc.VectorSubcoreMesh(core_axis_name="c", subcore_axis_name="s")
    run = functools.partial(
        pl.kernel,
        mesh=mesh,
        out_type=(
            jax.ShapeDtypeStruct((2, 2048, 1024), jnp.float32),
            jax.ShapeDtypeStruct((_NW, _RPW * 2 // 16, 16), jnp.float32),
            jax.ShapeDtypeStruct((1, 2048, 1024), jnp.float32),
        ),
        scratch_types=[
            pltpu.VMEM((2, _DR, 1024), jnp.float32),
            pltpu.VMEM((2, _YC, 128), jnp.float32),
            pltpu.VMEM((_RPW * 2 // 16, 16), jnp.float32),
            pltpu.SemaphoreType.DMA((2,)),
            pltpu.SemaphoreType.DMA((2,)),
            pltpu.SemaphoreType.DMA,
            pltpu.SemaphoreType.DMA,
        ],
        compiler_params=pltpu.CompilerParams(needs_layout_passes=False),
    )(_body)
    x_out, y_out, z_out = run(x2, y2, z2)
    return (x_out, y_out.reshape(4, 4096, 2), z_out)
